# Initial kernel scaffold; baseline (speedup 1.0000x reference)
import jax
import jax.numpy as jnp
from jax.experimental import pallas as pl


def kernel(x, ve_tables, input_ids):
    raise NotImplementedError("write your pallas kernel here")


# speedup vs baseline: 4.5131x; 4.5131x over previous
import functools

import jax
import jax.numpy as jnp
from jax import lax
from jax.experimental import pallas as pl
from jax.experimental.pallas import tpu as pltpu
from jax.experimental.pallas import tpu_sc as plsc

NUM_VE = 6
VOCAB = 100000
KV_DIM = 128
NIDS = 4 * 2048
NUM_CORES = 2
NUM_SUBCORES = 16
NW = NUM_CORES * NUM_SUBCORES
IDS_PER_W = NIDS // NW
CHUNK = 128
NCHUNK = IDS_PER_W // CHUNK
LANES = 16


def _make_kernel():
    mesh = plsc.VectorSubcoreMesh(core_axis_name="c", subcore_axis_name="s")

    @functools.partial(
        pl.kernel,
        mesh=mesh,
        out_type=jax.ShapeDtypeStruct((NUM_VE * NIDS, KV_DIM), jnp.float32),
        scratch_types=[
            pltpu.VMEM((IDS_PER_W,), jnp.int32),
            pltpu.VMEM((NCHUNK, CHUNK), jnp.int32),
            pltpu.VMEM((IDS_PER_W, KV_DIM), jnp.float32),
            pltpu.SemaphoreType.DMA,
        ],
    )
    def ve_gather(tables_hbm, ids_hbm, out_hbm, ids_v, idx_v, rows_v, sem):
        wid = lax.axis_index("s") * NUM_CORES + lax.axis_index("c")
        base = wid * IDS_PER_W
        pltpu.sync_copy(ids_hbm.at[pl.ds(base, IDS_PER_W)], ids_v)
        for layer in range(NUM_VE):
            off = layer * VOCAB
            for j in range(NCHUNK):
                for i in range(CHUNK // LANES):
                    idx_v[j, pl.ds(i * LANES, LANES)] = (
                        ids_v[pl.ds(j * CHUNK + i * LANES, LANES)] + off
                    )
            copies = [
                pltpu.async_copy(
                    tables_hbm.at[idx_v.at[j]],
                    rows_v.at[pl.ds(j * CHUNK, CHUNK)],
                    sem,
                )
                for j in range(NCHUNK)
            ]
            for c in copies:
                c.wait()
            pltpu.sync_copy(
                rows_v,
                out_hbm.at[pl.ds(layer * NIDS + base, IDS_PER_W)],
            )

    return ve_gather


_ve_gather = _make_kernel()


def kernel(x, ve_tables, input_ids):
    tables_flat = ve_tables.reshape(NUM_VE * VOCAB, KV_DIM)
    ids_flat = input_ids.reshape(NIDS)
    out = _ve_gather(tables_flat, ids_flat)
    B, T = input_ids.shape
    return out.reshape(NUM_VE, B, T, KV_DIM).astype(x.dtype)


# speedup vs baseline: 4.8932x; 1.0842x over previous
import functools

import jax
import jax.numpy as jnp
from jax import lax
from jax.experimental import pallas as pl
from jax.experimental.pallas import tpu as pltpu
from jax.experimental.pallas import tpu_sc as plsc

NUM_VE = 6
VOCAB = 100000
KV_DIM = 128
NIDS = 4 * 2048
NUM_CORES = 2
NUM_SUBCORES = 16
NW = NUM_CORES * NUM_SUBCORES
IDS_PER_W = NIDS // NW
CHUNK = 128
NCHUNK = IDS_PER_W // CHUNK
LANES = 16


def _make_kernel():
    mesh = plsc.VectorSubcoreMesh(core_axis_name="c", subcore_axis_name="s")

    @functools.partial(
        pl.kernel,
        mesh=mesh,
        out_type=jax.ShapeDtypeStruct((NUM_VE * NIDS, KV_DIM), jnp.float32),
        scratch_types=[
            pltpu.VMEM((IDS_PER_W,), jnp.int32),
            pltpu.VMEM((NUM_VE * NCHUNK, CHUNK), jnp.int32),
            pltpu.VMEM((IDS_PER_W, KV_DIM), jnp.float32),
            pltpu.VMEM((IDS_PER_W, KV_DIM), jnp.float32),
            pltpu.SemaphoreType.DMA,
            pltpu.SemaphoreType.DMA,
            pltpu.SemaphoreType.DMA,
            pltpu.SemaphoreType.DMA,
        ],
    )
    def ve_gather(tables_hbm, ids_hbm, out_hbm, ids_v, idx_v,
                  rows0, rows1, sg0, sg1, sw0, sw1):
        wid = lax.axis_index("s") * NUM_CORES + lax.axis_index("c")
        base = wid * IDS_PER_W
        pltpu.sync_copy(ids_hbm.at[pl.ds(base, IDS_PER_W)], ids_v)
        for layer in range(NUM_VE):
            off = layer * VOCAB
            for j in range(NCHUNK):
                for i in range(CHUNK // LANES):
                    idx_v[layer * NCHUNK + j, pl.ds(i * LANES, LANES)] = (
                        ids_v[pl.ds(j * CHUNK + i * LANES, LANES)] + off
                    )
        rows = [rows0, rows1]
        sg = [sg0, sg1]
        sw = [sw0, sw1]
        pend_g = [None, None]
        pend_w = [None, None]

        def fire_gather(layer):
            b = layer & 1
            pend_g[b] = [
                pltpu.async_copy(
                    tables_hbm.at[idx_v.at[layer * NCHUNK + j]],
                    rows[b].at[pl.ds(j * CHUNK, CHUNK)],
                    sg[b],
                )
                for j in range(NCHUNK)
            ]

        fire_gather(0)
        for layer in range(NUM_VE):
            b = layer & 1
            if layer + 1 < NUM_VE:
                nb = (layer + 1) & 1
                if pend_w[nb] is not None:
                    pend_w[nb].wait()
                    pend_w[nb] = None
                fire_gather(layer + 1)
            for c in pend_g[b]:
                c.wait()
            pend_w[b] = pltpu.async_copy(
                rows[b],
                out_hbm.at[pl.ds(layer * NIDS + base, IDS_PER_W)],
                sw[b],
            )
        for b in range(2):
            if pend_w[b] is not None:
                pend_w[b].wait()

    return ve_gather


_ve_gather = _make_kernel()


def kernel(x, ve_tables, input_ids):
    tables_flat = ve_tables.reshape(NUM_VE * VOCAB, KV_DIM)
    ids_flat = input_ids.reshape(NIDS)
    out = _ve_gather(tables_flat, ids_flat)
    B, T = input_ids.shape
    return out.reshape(NUM_VE, B, T, KV_DIM).astype(x.dtype)


# speedup vs baseline: 4.8936x; 1.0001x over previous
import functools

import jax
import jax.numpy as jnp
from jax import lax
from jax.experimental import pallas as pl
from jax.experimental.pallas import tpu as pltpu
from jax.experimental.pallas import tpu_sc as plsc

NUM_VE = 6
VOCAB = 100000
KV_DIM = 128
NIDS = 4 * 2048
NUM_CORES = 2
NUM_SUBCORES = 16
NW = NUM_CORES * NUM_SUBCORES
IDS_PER_W = NIDS // NW
CHUNK = 128
NCHUNK = IDS_PER_W // CHUNK
NCHUNKS_TOTAL = NUM_VE * NCHUNK
RING = 4
LANES = 16


def _make_kernel():
    mesh = plsc.VectorSubcoreMesh(core_axis_name="c", subcore_axis_name="s")

    @functools.partial(
        pl.kernel,
        mesh=mesh,
        out_type=jax.ShapeDtypeStruct((NUM_VE * NIDS, KV_DIM), jnp.float32),
        scratch_types=[
            pltpu.VMEM((IDS_PER_W,), jnp.int32),
            pltpu.VMEM((NCHUNKS_TOTAL, CHUNK), jnp.int32),
        ]
        + [pltpu.VMEM((CHUNK, KV_DIM), jnp.float32) for _ in range(RING)]
        + [pltpu.SemaphoreType.DMA for _ in range(2 * RING)],
    )
    def ve_gather(tables_hbm, ids_hbm, out_hbm, ids_v, idx_v, *bufs_and_sems):
        rows = list(bufs_and_sems[:RING])
        sg = list(bufs_and_sems[RING:2 * RING])
        sw = list(bufs_and_sems[2 * RING:])
        wid = lax.axis_index("s") * NUM_CORES + lax.axis_index("c")
        base = wid * IDS_PER_W
        pltpu.sync_copy(ids_hbm.at[pl.ds(base, IDS_PER_W)], ids_v)

        def idx_body(layer, _):
            off = layer * VOCAB
            for j in range(NCHUNK):
                for i in range(CHUNK // LANES):
                    idx_v[layer * NCHUNK + j, pl.ds(i * LANES, LANES)] = (
                        ids_v[pl.ds(j * CHUNK + i * LANES, LANES)] + off
                    )
            return _
        lax.fori_loop(0, NUM_VE, idx_body, 0, unroll=False)

        def fire_gather(c, b):
            pltpu.async_copy(tables_hbm.at[idx_v.at[c]], rows[b], sg[b])

        def wait_gather(b):
            pltpu.make_async_copy(
                tables_hbm.at[pl.ds(0, CHUNK)], rows[b], sg[b]).wait()

        def out_off(c):
            return (c // NCHUNK) * NIDS + (c % NCHUNK) * CHUNK + base

        def fire_write(c, b):
            pltpu.async_copy(rows[b], out_hbm.at[pl.ds(out_off(c), CHUNK)], sw[b])

        def wait_write(b):
            pltpu.make_async_copy(
                rows[b], out_hbm.at[pl.ds(0, CHUNK)], sw[b]).wait()

        for b in range(RING):
            fire_gather(b, b)

        def ring_body(g, carry):
            for b in range(RING):
                c = g * RING + b
                wait_gather(b)
                fire_write(c, b)

                @pl.when(g < NCHUNKS_TOTAL // RING - 1)
                def _refill():
                    wait_write(b)
                    fire_gather(c + RING, b)
            return carry
        lax.fori_loop(0, NCHUNKS_TOTAL // RING, ring_body, 0, unroll=False)

        for b in range(RING):
            wait_write(b)

    return ve_gather


_ve_gather = _make_kernel()


def kernel(x, ve_tables, input_ids):
    tables_flat = ve_tables.reshape(NUM_VE * VOCAB, KV_DIM)
    ids_flat = input_ids.reshape(NIDS)
    out = _ve_gather(tables_flat, ids_flat)
    B, T = input_ids.shape
    return out.reshape(NUM_VE, B, T, KV_DIM).astype(x.dtype)


# speedup vs baseline: 5.1022x; 1.0426x over previous
import functools

import jax
import jax.numpy as jnp
from jax import lax
from jax.experimental import pallas as pl
from jax.experimental.pallas import tpu as pltpu
from jax.experimental.pallas import tpu_sc as plsc

NUM_VE = 6
VOCAB = 100000
KV_DIM = 128
NIDS = 4 * 2048
NUM_CORES = 2
NUM_SUBCORES = 16
NW = NUM_CORES * NUM_SUBCORES
IDS_PER_W = NIDS // NW
CHUNK = 128
NCHUNK = IDS_PER_W // CHUNK
NCHUNKS_TOTAL = NUM_VE * NCHUNK
RING = 6
LANES = 16


def _make_kernel():
    mesh = plsc.VectorSubcoreMesh(core_axis_name="c", subcore_axis_name="s")

    @functools.partial(
        pl.kernel,
        mesh=mesh,
        out_type=jax.ShapeDtypeStruct((NUM_VE * NIDS, KV_DIM), jnp.float32),
        scratch_types=[
            pltpu.VMEM((IDS_PER_W,), jnp.int32),
            pltpu.VMEM((NCHUNKS_TOTAL, CHUNK), jnp.int32),
        ]
        + [pltpu.VMEM((CHUNK, KV_DIM), jnp.float32) for _ in range(RING)]
        + [pltpu.SemaphoreType.DMA for _ in range(2 * RING)],
    )
    def ve_gather(tables_hbm, ids_hbm, out_hbm, ids_v, idx_v, *bufs_and_sems):
        rows = list(bufs_and_sems[:RING])
        sg = list(bufs_and_sems[RING:2 * RING])
        sw = list(bufs_and_sems[2 * RING:])
        wid = lax.axis_index("s") * NUM_CORES + lax.axis_index("c")
        base = wid * IDS_PER_W
        pltpu.sync_copy(ids_hbm.at[pl.ds(base, IDS_PER_W)], ids_v)

        def compute_idx(layer):
            off = layer * VOCAB
            for j in range(NCHUNK):
                for i in range(CHUNK // LANES):
                    idx_v[layer * NCHUNK + j, pl.ds(i * LANES, LANES)] = (
                        ids_v[pl.ds(j * CHUNK + i * LANES, LANES)] + off
                    )

        def fire_gather(c, b):
            pltpu.async_copy(tables_hbm.at[idx_v.at[c]], rows[b], sg[b])

        def wait_gather(b):
            pltpu.make_async_copy(
                tables_hbm.at[pl.ds(0, CHUNK)], rows[b], sg[b]).wait()

        def out_off(c):
            return (c // NCHUNK) * NIDS + (c % NCHUNK) * CHUNK + base

        def fire_write(c, b):
            pltpu.async_copy(rows[b], out_hbm.at[pl.ds(out_off(c), CHUNK)], sw[b])

        def wait_write(b):
            pltpu.make_async_copy(
                rows[b], out_hbm.at[pl.ds(0, CHUNK)], sw[b]).wait()

        for layer in range(RING // NCHUNK):
            compute_idx(layer)
            for j in range(NCHUNK):
                c = layer * NCHUNK + j
                fire_gather(c, c)

        def idx_body(layer, carry):
            compute_idx_dyn(layer)
            return carry

        def compute_idx_dyn(layer):
            compute_idx(layer)

        lax.fori_loop(RING // NCHUNK, NUM_VE, idx_body, 0, unroll=False)

        def ring_body(g, carry):
            for b in range(RING):
                c = g * RING + b
                wait_gather(b)
                fire_write(c, b)

                @pl.when(g < NCHUNKS_TOTAL // RING - 1)
                def _refill():
                    wait_write(b)
                    fire_gather(c + RING, b)
            return carry
        lax.fori_loop(0, NCHUNKS_TOTAL // RING, ring_body, 0, unroll=False)

        for b in range(RING):
            wait_write(b)

    return ve_gather


_ve_gather = _make_kernel()


def kernel(x, ve_tables, input_ids):
    tables_flat = ve_tables.reshape(NUM_VE * VOCAB, KV_DIM)
    ids_flat = input_ids.reshape(NIDS)
    out = _ve_gather(tables_flat, ids_flat)
    B, T = input_ids.shape
    return out.reshape(NUM_VE, B, T, KV_DIM).astype(x.dtype)
